# baseline (speedup 1.0000x reference)
import jax
import jax.numpy as jnp
from jax.experimental import pallas as pl
from jax.experimental.pallas import tpu as pltpu
from functools import partial


def _square_distance(src, dst):
    d = -2.0 * jnp.einsum('bsc,bnc->bsn', src, dst)
    d = d + jnp.sum(src ** 2, axis=-1)[:, :, None]
    d = d + jnp.sum(dst ** 2, axis=-1)[:, None, :]
    return d


def _index_points(points, idx):
    return jax.vmap(lambda p, i: p[i])(points, idx)


def _fps(xyz, npoint):
    B, N, _ = xyz.shape
    def body(i, state):
        centroids, distance, farthest = state
        centroids = centroids.at[:, i].set(farthest)
        centroid = jax.vmap(lambda p, j: p[j])(xyz, farthest)[:, None, :]
        dist = jnp.sum((xyz - centroid) ** 2, axis=-1)
        distance = jnp.minimum(distance, dist)
        farthest = jnp.argmax(distance, axis=-1).astype(jnp.int32)
        return (centroids, distance, farthest)
    init = (jnp.zeros((B, npoint), dtype=jnp.int32),
            jnp.full((B, N), 1e10, dtype=xyz.dtype),
            jnp.zeros((B,), dtype=jnp.int32))
    centroids, _, _ = jax.lax.fori_loop(0, npoint, body, init)
    return centroids


def _query_ball(radius, nsample, xyz, new_xyz):
    B, N, _ = xyz.shape
    S = new_xyz.shape[1]
    sqrdists = _square_distance(new_xyz, xyz)
    group_idx = jnp.broadcast_to(jnp.arange(N, dtype=jnp.int32), (B, S, N))
    group_idx = jnp.where(sqrdists > radius ** 2, N, group_idx)
    group_idx = jnp.sort(group_idx, axis=-1)[:, :, :nsample]
    group_first = jnp.broadcast_to(group_idx[:, :, :1], group_idx.shape)
    group_idx = jnp.where(group_idx == N, group_first, group_idx)
    return group_idx


def _sample_and_group(npoint, radius, nsample, xyz, points):
    fps_idx = _fps(xyz, npoint)
    new_xyz = _index_points(xyz, fps_idx)
    idx = _query_ball(radius, nsample, xyz, new_xyz)
    grouped_xyz = _index_points(xyz, idx)
    grouped_xyz_norm = grouped_xyz - new_xyz[:, :, None, :]
    if points is not None:
        grouped_points = _index_points(points, idx)
        new_points = jnp.concatenate([grouped_xyz_norm, grouped_points], axis=-1)
    else:
        new_points = grouped_xyz_norm
    return new_xyz, new_points


def _sa_mlp(new_points, Ws, bs):
    for W, b in zip(Ws, bs):
        new_points = jax.nn.relu(new_points @ W + b)
    return jnp.max(new_points, axis=2)


def _tail_kernel(xyz2_ref, p2_ref, W3_0_ref, b3_0_ref, W3_1_ref, b3_1_ref,
                 Wd1_ref, bd1_ref, Wd3_ref, bd3_ref, out_ref):
    B, S, _ = p2_ref.shape
    feats = jnp.concatenate([xyz2_ref[...], p2_ref[...]], axis=-1)
    h = feats.reshape(B * S, feats.shape[-1])
    h = jnp.maximum(jnp.dot(h, W3_0_ref[...],
                            preferred_element_type=jnp.float32) + b3_0_ref[...], 0.0)
    h = jnp.maximum(jnp.dot(h, W3_1_ref[...],
                            preferred_element_type=jnp.float32) + b3_1_ref[...], 0.0)
    h = jnp.max(h.reshape(B, S, -1), axis=1)
    h = jnp.maximum(jnp.dot(h, Wd1_ref[...],
                            preferred_element_type=jnp.float32) + bd1_ref[...], 0.0)
    z = jnp.dot(h, Wd3_ref[...], preferred_element_type=jnp.float32) + bd3_ref[...]
    out_ref[...] = jax.nn.sigmoid(z)


def kernel(input, W1_0, b1_0, W1_1, b1_1, W2_0, b2_0, W2_1, b2_1,
           W3_0, b3_0, W3_1, b3_1, Wd1, bd1, Wd3, bd3):
    x = input
    B = x.shape[0]
    xyz1, np1 = _sample_and_group(256, 0.1, 32, x, None)
    p1 = _sa_mlp(np1, [W1_0, W1_1], [b1_0, b1_1])
    xyz2, np2 = _sample_and_group(128, 0.2, 64, xyz1, p1)
    p2 = _sa_mlp(np2, [W2_0, W2_1], [b2_0, b2_1])

    pred = pl.pallas_call(
        _tail_kernel,
        out_shape=jax.ShapeDtypeStruct((B, 1), jnp.float32),
    )(xyz2, p2, W3_0, b3_0, W3_1, b3_1, Wd1, bd1, Wd3, bd3)
    return pred


# speedup vs baseline: 1.7495x; 1.7495x over previous
import jax
import jax.numpy as jnp
from jax import lax
from jax.experimental import pallas as pl
from jax.experimental.pallas import tpu as pltpu
from jax.experimental.pallas import tpu_sc as plsc
from functools import partial


def _fps(xyz, npoint):
    B, N, _ = xyz.shape
    def body(i, state):
        centroids, distance, farthest = state
        centroids = centroids.at[:, i].set(farthest)
        centroid = jax.vmap(lambda p, j: p[j])(xyz, farthest)[:, None, :]
        dist = jnp.sum((xyz - centroid) ** 2, axis=-1)
        distance = jnp.minimum(distance, dist)
        farthest = jnp.argmax(distance, axis=-1).astype(jnp.int32)
        return (centroids, distance, farthest)
    init = (jnp.zeros((B, npoint), dtype=jnp.int32),
            jnp.full((B, N), 1e10, dtype=xyz.dtype),
            jnp.zeros((B,), dtype=jnp.int32))
    centroids, _, _ = jax.lax.fori_loop(0, npoint, body, init)
    return centroids


def _index_points(points, idx):
    return jax.vmap(lambda p, i: p[i])(points, idx)


def _sa_mlp(new_points, Ws, bs):
    for W, b in zip(Ws, bs):
        new_points = jax.nn.relu(new_points @ W + b)
    return jnp.max(new_points, axis=2)


def _ball_group_sc(xyzt, cent, radius, nsample, n_chunks, s_half):
    B3N = xyzt.shape[0]
    B3S = cent.shape[0]
    N = n_chunks * 16
    S = 2 * s_half
    B = B3N // (3 * N)
    r2 = jnp.float32(float(radius) ** 2)
    cap = 2 * nsample
    mesh = plsc.VectorSubcoreMesh(core_axis_name="c", subcore_axis_name="s")

    @partial(
        pl.kernel,
        mesh=mesh,
        compiler_params=pltpu.CompilerParams(needs_layout_passes=False),
        out_type=jax.ShapeDtypeStruct((B * S * 3 * nsample,), jnp.float32),
        scratch_types=[
            pltpu.VMEM((N,), jnp.float32),
            pltpu.VMEM((N,), jnp.float32),
            pltpu.VMEM((N,), jnp.float32),
            pltpu.VMEM((N,), jnp.float32),
            pltpu.VMEM((S,), jnp.float32),
            pltpu.VMEM((S,), jnp.float32),
            pltpu.VMEM((S,), jnp.float32),
            pltpu.VMEM((cap,), jnp.float32),
            pltpu.VMEM((cap,), jnp.float32),
            pltpu.VMEM((cap,), jnp.float32),
            pltpu.VMEM((s_half * 3 * nsample,), jnp.float32),
        ],
    )
    def k(xyzt_hbm, cent_hbm, out_hbm, x_v, y_v, z_v, xn_v,
          cx_v, cy_v, cz_v, gx_b, gy_b, gz_b, g_v):
        cid = lax.axis_index("c")
        sid = lax.axis_index("s")
        wid = sid * 2 + cid
        b = wid // 2
        half = wid % 2
        pltpu.sync_copy(xyzt_hbm.at[pl.ds(b * 3 * N, N)], x_v)
        pltpu.sync_copy(xyzt_hbm.at[pl.ds(b * 3 * N + N, N)], y_v)
        pltpu.sync_copy(xyzt_hbm.at[pl.ds(b * 3 * N + 2 * N, N)], z_v)
        pltpu.sync_copy(cent_hbm.at[pl.ds(b * 3 * S, S)], cx_v)
        pltpu.sync_copy(cent_hbm.at[pl.ds(b * 3 * S + S, S)], cy_v)
        pltpu.sync_copy(cent_hbm.at[pl.ds(b * 3 * S + 2 * S, S)], cz_v)

        iota16 = lax.iota(jnp.int32, 16)
        zero16 = jnp.zeros((16,), jnp.int32)
        one16 = jnp.full((16,), 1, jnp.int32)
        two16 = jnp.full((16,), 2, jnp.int32)

        def norm_fn(i, carry):
            base = i * 16
            xv = x_v[pl.ds(base, 16)]
            yv = y_v[pl.ds(base, 16)]
            zv = z_v[pl.ds(base, 16)]
            xn_v[pl.ds(base, 16)] = xv * xv + yv * yv + zv * zv
            return carry
        lax.fori_loop(0, n_chunks, norm_fn, jnp.int32(0))

        def t2_row(s_local, carry):
            s_glob = half * s_half + s_local
            sidx = jnp.full((16,), s_glob, jnp.int32)
            cx = plsc.load_gather(cx_v, [sidx])
            cy = plsc.load_gather(cy_v, [sidx])
            cz = plsc.load_gather(cz_v, [sidx])
            cn = cx * cx + cy * cy + cz * cz
            iota16_ = lax.iota(jnp.int32, 16)

            def chunk_fn(i, kcnt):
                base = i * 16
                xv = x_v[pl.ds(base, 16)]
                yv = y_v[pl.ds(base, 16)]
                zv = z_v[pl.ds(base, 16)]
                nv = xn_v[pl.ds(base, 16)]
                d = -2.0 * (xv * cx + yv * cy + zv * cz) + nv + cn
                m = d <= r2
                ones = jnp.where(m, 1, 0).astype(jnp.int32)
                csum = plsc.cumsum(ones)
                tgt = kcnt + csum - 1
                mm = m & (tgt < cap - 16)
                plsc.store_scatter(gx_b, [tgt], xv - cx, mask=mm)
                plsc.store_scatter(gy_b, [tgt], yv - cy, mask=mm)
                plsc.store_scatter(gz_b, [tgt], zv - cz, mask=mm)
                return kcnt + jnp.sum(ones)

            kcnt = lax.fori_loop(0, n_chunks, chunk_fn, jnp.int32(0))
            gbase = s_local * 3 * nsample
            zidx = jnp.zeros((16,), jnp.int32)
            g0x = plsc.load_gather(gx_b, [zidx])
            g0y = plsc.load_gather(gy_b, [zidx])
            g0z = plsc.load_gather(gz_b, [zidx])
            for cc in range(nsample // 16):
                pos = iota16_ + (cc * 16)
                keep = pos < kcnt
                vx = jnp.where(keep, gx_b[pl.ds(cc * 16, 16)], g0x)
                vy = jnp.where(keep, gy_b[pl.ds(cc * 16, 16)], g0y)
                vz = jnp.where(keep, gz_b[pl.ds(cc * 16, 16)], g0z)
                g_v[pl.ds(gbase + cc * 16, 16)] = vx
                g_v[pl.ds(gbase + nsample + cc * 16, 16)] = vy
                g_v[pl.ds(gbase + 2 * nsample + cc * 16, 16)] = vz
            return carry
        lax.fori_loop(0, s_half, t2_row, jnp.int32(0))
        pltpu.sync_copy(
            g_v, out_hbm.at[pl.ds((b * S + half * s_half) * 3 * nsample,
                                  s_half * 3 * nsample)])
        return

        def row_fn(s_local, carry):
            s_glob = half * s_half + s_local
            sidx = jnp.full((16,), s_glob, jnp.int32)
            cx = plsc.load_gather(cx_v, [sidx])
            cy = plsc.load_gather(cy_v, [sidx])
            cz = plsc.load_gather(cz_v, [sidx])
            cn = cx * cx + cy * cy + cz * cz

            def chunk_fn(i, kcnt):
                base = i * 16
                xv = x_v[pl.ds(base, 16)]
                yv = y_v[pl.ds(base, 16)]
                zv = z_v[pl.ds(base, 16)]
                nv = xn_v[pl.ds(base, 16)]
                d = -2.0 * (xv * cx + yv * cy + zv * cz) + nv + cn
                m = d <= r2
                ones = jnp.where(m, 1, 0).astype(jnp.int32)
                csum = plsc.cumsum(ones)
                tgt = kcnt + csum - 1
                mm = m & (tgt < cap - 16)
                plsc.store_scatter(idx_v, [tgt], base + iota16, mask=mm)
                return kcnt + jnp.sum(ones)

            kcnt = lax.fori_loop(0, n_chunks, chunk_fn, jnp.int32(0))

            i0 = plsc.load_gather(idx_v, [zero16])
            gbase = s_local * 3 * nsample
            for cc in range(nsample // 16):
                v = idx_v[pl.ds(cc * 16, 16)]
                pos = iota16 + (cc * 16)
                v = jnp.where(pos < kcnt, v, i0)
                gx = plsc.load_gather(x_v, [v])
                gy = plsc.load_gather(y_v, [v])
                gz = plsc.load_gather(z_v, [v])
                g_v[pl.ds(gbase + cc * 16, 16)] = gx - cx
                g_v[pl.ds(gbase + nsample + cc * 16, 16)] = gy - cy
                g_v[pl.ds(gbase + 2 * nsample + cc * 16, 16)] = gz - cz
            return carry
        lax.fori_loop(0, s_half, row_fn, jnp.int32(0))
        pltpu.sync_copy(
            g_v, out_hbm.at[pl.ds((b * S + half * s_half) * 3 * nsample,
                                  s_half * 3 * nsample)])

    return k(xyzt, cent)


def _query_ball_ref(radius, nsample, xyz, new_xyz):
    B, N, _ = xyz.shape
    S = new_xyz.shape[1]
    d = -2.0 * jnp.einsum('bsc,bnc->bsn', new_xyz, xyz)
    d = d + jnp.sum(new_xyz ** 2, axis=-1)[:, :, None]
    d = d + jnp.sum(xyz ** 2, axis=-1)[:, None, :]
    group_idx = jnp.broadcast_to(jnp.arange(N, dtype=jnp.int32), (B, S, N))
    group_idx = jnp.where(d > radius ** 2, N, group_idx)
    group_idx = jnp.sort(group_idx, axis=-1)[:, :, :nsample]
    group_first = jnp.broadcast_to(group_idx[:, :, :1], group_idx.shape)
    group_idx = jnp.where(group_idx == N, group_first, group_idx)
    return group_idx


def _tail_kernel(xyz2_ref, p2_ref, W3_0_ref, b3_0_ref, W3_1_ref, b3_1_ref,
                 Wd1_ref, bd1_ref, Wd3_ref, bd3_ref, out_ref):
    B, S, _ = p2_ref.shape
    feats = jnp.concatenate([xyz2_ref[...], p2_ref[...]], axis=-1)
    h = feats.reshape(B * S, feats.shape[-1])
    h = jnp.maximum(jnp.dot(h, W3_0_ref[...],
                            preferred_element_type=jnp.float32) + b3_0_ref[...], 0.0)
    h = jnp.maximum(jnp.dot(h, W3_1_ref[...],
                            preferred_element_type=jnp.float32) + b3_1_ref[...], 0.0)
    h = jnp.max(h.reshape(B, S, -1), axis=1)
    h = jnp.maximum(jnp.dot(h, Wd1_ref[...],
                            preferred_element_type=jnp.float32) + bd1_ref[...], 0.0)
    z = jnp.dot(h, Wd3_ref[...], preferred_element_type=jnp.float32) + bd3_ref[...]
    out_ref[...] = jax.nn.sigmoid(z)


def kernel(input, W1_0, b1_0, W1_1, b1_1, W2_0, b2_0, W2_1, b2_1,
           W3_0, b3_0, W3_1, b3_1, Wd1, bd1, Wd3, bd3):
    x = input
    B, N, _ = x.shape

    fps_idx = _fps(x, 256)
    new_xyz = _index_points(x, fps_idx)
    xyzt = jnp.transpose(x, (0, 2, 1)).reshape(-1)
    cent = jnp.transpose(new_xyz, (0, 2, 1)).reshape(-1)
    g1 = _ball_group_sc(xyzt, cent, 0.1, 32, N // 16, 128)
    g1 = g1.reshape(B, 256, 3, 32)
    np1 = jnp.transpose(g1, (0, 1, 3, 2))
    p1 = _sa_mlp(np1, [W1_0, W1_1], [b1_0, b1_1])

    fps_idx2 = _fps(new_xyz, 128)
    xyz2 = _index_points(new_xyz, fps_idx2)
    idx2 = _query_ball_ref(0.2, 64, new_xyz, xyz2)
    g2 = _index_points(new_xyz, idx2) - xyz2[:, :, None, :]
    gp2 = _index_points(p1, idx2)
    np2 = jnp.concatenate([g2, gp2], axis=-1)
    p2 = _sa_mlp(np2, [W2_0, W2_1], [b2_0, b2_1])

    pred = pl.pallas_call(
        _tail_kernel,
        out_shape=jax.ShapeDtypeStruct((B, 1), jnp.float32),
    )(xyz2, p2, W3_0, b3_0, W3_1, b3_1, Wd1, bd1, Wd3, bd3)
    return pred


# speedup vs baseline: 5.0447x; 2.8835x over previous
import jax
import jax.numpy as jnp
from jax import lax
from jax.experimental import pallas as pl
from jax.experimental.pallas import tpu as pltpu
from jax.experimental.pallas import tpu_sc as plsc
from functools import partial


def _fps(xyz, npoint):
    B, N, _ = xyz.shape
    def body(i, state):
        centroids, distance, farthest = state
        centroids = centroids.at[:, i].set(farthest)
        centroid = jax.vmap(lambda p, j: p[j])(xyz, farthest)[:, None, :]
        dist = jnp.sum((xyz - centroid) ** 2, axis=-1)
        distance = jnp.minimum(distance, dist)
        farthest = jnp.argmax(distance, axis=-1).astype(jnp.int32)
        return (centroids, distance, farthest)
    init = (jnp.zeros((B, npoint), dtype=jnp.int32),
            jnp.full((B, N), 1e10, dtype=xyz.dtype),
            jnp.zeros((B,), dtype=jnp.int32))
    centroids, _, _ = jax.lax.fori_loop(0, npoint, body, init)
    return centroids


def _index_points(points, idx):
    return jax.vmap(lambda p, i: p[i])(points, idx)


def _fps_kernel_body(npoint, x_ref, y_ref, z_ref, cx_ref, cy_ref, cz_ref,
                     dist_ref):
    B, N = x_ref.shape
    xr = x_ref[...]
    yr = y_ref[...]
    zr = z_ref[...]
    iota_n = lax.broadcasted_iota(jnp.int32, (B, N), 1)
    iota_p = lax.broadcasted_iota(jnp.int32, (B, npoint), 1)
    dist_ref[...] = jnp.full((B, N), 1e10, jnp.float32)

    def step(i, carry):
        far, cxa, cya, cza = carry
        onehot = iota_n == far
        fx = jnp.sum(jnp.where(onehot, xr, 0.0), axis=1, keepdims=True)
        fy = jnp.sum(jnp.where(onehot, yr, 0.0), axis=1, keepdims=True)
        fz = jnp.sum(jnp.where(onehot, zr, 0.0), axis=1, keepdims=True)
        slot = iota_p == i
        cxa = jnp.where(slot, fx, cxa)
        cya = jnp.where(slot, fy, cya)
        cza = jnp.where(slot, fz, cza)
        d = (xr - fx) ** 2 + (yr - fy) ** 2 + (zr - fz) ** 2
        dist = jnp.minimum(dist_ref[...], d)
        dist_ref[...] = dist
        mx = jnp.max(dist, axis=1, keepdims=True)
        idx = jnp.min(jnp.where(dist == mx, iota_n, N), axis=1, keepdims=True)
        return (idx.astype(jnp.int32), cxa, cya, cza)

    zc = jnp.zeros((B, npoint), jnp.float32)
    _, cxa, cya, cza = lax.fori_loop(
        0, npoint, step, (jnp.zeros((B, 1), jnp.int32), zc, zc, zc))
    cx_ref[...] = cxa
    cy_ref[...] = cya
    cz_ref[...] = cza


def _fps_tc(xb, yb, zb, npoint):
    B, N = xb.shape
    out = pl.pallas_call(
        partial(_fps_kernel_body, npoint),
        out_shape=[jax.ShapeDtypeStruct((B, npoint), jnp.float32)] * 3,
        scratch_shapes=[pltpu.VMEM((B, N), jnp.float32)],
    )(xb, yb, zb)
    return out


def _sa_mlp(new_points, Ws, bs):
    for W, b in zip(Ws, bs):
        new_points = jax.nn.relu(new_points @ W + b)
    return jnp.max(new_points, axis=2)


def _ball_group_sc(xyzt, cent, radius, nsample, n_chunks, s_half):
    B3N = xyzt.shape[0]
    B3S = cent.shape[0]
    N = n_chunks * 16
    S = 2 * s_half
    B = B3N // (3 * N)
    r2 = jnp.float32(float(radius) ** 2)
    cap = 2 * nsample
    mesh = plsc.VectorSubcoreMesh(core_axis_name="c", subcore_axis_name="s")

    @partial(
        pl.kernel,
        mesh=mesh,
        compiler_params=pltpu.CompilerParams(needs_layout_passes=False),
        out_type=jax.ShapeDtypeStruct((B * S * 3 * nsample,), jnp.float32),
        scratch_types=[
            pltpu.VMEM((N,), jnp.float32),
            pltpu.VMEM((N,), jnp.float32),
            pltpu.VMEM((N,), jnp.float32),
            pltpu.VMEM((N,), jnp.float32),
            pltpu.VMEM((S,), jnp.float32),
            pltpu.VMEM((S,), jnp.float32),
            pltpu.VMEM((S,), jnp.float32),
            pltpu.VMEM((cap,), jnp.float32),
            pltpu.VMEM((cap,), jnp.float32),
            pltpu.VMEM((cap,), jnp.float32),
            pltpu.VMEM((s_half * 3 * nsample,), jnp.float32),
        ],
    )
    def k(xyzt_hbm, cent_hbm, out_hbm, x_v, y_v, z_v, xn_v,
          cx_v, cy_v, cz_v, gx_b, gy_b, gz_b, g_v):
        cid = lax.axis_index("c")
        sid = lax.axis_index("s")
        wid = sid * 2 + cid
        b = wid // 2
        half = wid % 2
        pltpu.sync_copy(xyzt_hbm.at[pl.ds(b * 3 * N, N)], x_v)
        pltpu.sync_copy(xyzt_hbm.at[pl.ds(b * 3 * N + N, N)], y_v)
        pltpu.sync_copy(xyzt_hbm.at[pl.ds(b * 3 * N + 2 * N, N)], z_v)
        pltpu.sync_copy(cent_hbm.at[pl.ds(b * 3 * S, S)], cx_v)
        pltpu.sync_copy(cent_hbm.at[pl.ds(b * 3 * S + S, S)], cy_v)
        pltpu.sync_copy(cent_hbm.at[pl.ds(b * 3 * S + 2 * S, S)], cz_v)

        iota16 = lax.iota(jnp.int32, 16)
        zero16 = jnp.zeros((16,), jnp.int32)
        one16 = jnp.full((16,), 1, jnp.int32)
        two16 = jnp.full((16,), 2, jnp.int32)

        def norm_fn(i, carry):
            base = i * 16
            xv = x_v[pl.ds(base, 16)]
            yv = y_v[pl.ds(base, 16)]
            zv = z_v[pl.ds(base, 16)]
            xn_v[pl.ds(base, 16)] = xv * xv + yv * yv + zv * zv
            return carry
        lax.fori_loop(0, n_chunks, norm_fn, jnp.int32(0))

        def t2_row(s_local, carry):
            s_glob = half * s_half + s_local
            sidx = jnp.full((16,), s_glob, jnp.int32)
            cx = plsc.load_gather(cx_v, [sidx])
            cy = plsc.load_gather(cy_v, [sidx])
            cz = plsc.load_gather(cz_v, [sidx])
            cn = cx * cx + cy * cy + cz * cz
            iota16_ = lax.iota(jnp.int32, 16)

            def chunk_fn(i, kcnt):
                base = i * 16
                xv = x_v[pl.ds(base, 16)]
                yv = y_v[pl.ds(base, 16)]
                zv = z_v[pl.ds(base, 16)]
                nv = xn_v[pl.ds(base, 16)]
                d = -2.0 * (xv * cx + yv * cy + zv * cz) + nv + cn
                m = d <= r2
                ones = jnp.where(m, 1, 0).astype(jnp.int32)
                csum = plsc.cumsum(ones)
                tgt = kcnt + csum - 1
                mm = m & (tgt < cap - 16)
                plsc.store_scatter(gx_b, [tgt], xv - cx, mask=mm)
                plsc.store_scatter(gy_b, [tgt], yv - cy, mask=mm)
                plsc.store_scatter(gz_b, [tgt], zv - cz, mask=mm)
                return kcnt + jnp.sum(ones)

            kcnt = lax.fori_loop(0, n_chunks, chunk_fn, jnp.int32(0))
            gbase = s_local * 3 * nsample
            zidx = jnp.zeros((16,), jnp.int32)
            g0x = plsc.load_gather(gx_b, [zidx])
            g0y = plsc.load_gather(gy_b, [zidx])
            g0z = plsc.load_gather(gz_b, [zidx])
            for cc in range(nsample // 16):
                pos = iota16_ + (cc * 16)
                keep = pos < kcnt
                vx = jnp.where(keep, gx_b[pl.ds(cc * 16, 16)], g0x)
                vy = jnp.where(keep, gy_b[pl.ds(cc * 16, 16)], g0y)
                vz = jnp.where(keep, gz_b[pl.ds(cc * 16, 16)], g0z)
                g_v[pl.ds(gbase + cc * 16, 16)] = vx
                g_v[pl.ds(gbase + nsample + cc * 16, 16)] = vy
                g_v[pl.ds(gbase + 2 * nsample + cc * 16, 16)] = vz
            return carry
        lax.fori_loop(0, s_half, t2_row, jnp.int32(0))
        pltpu.sync_copy(
            g_v, out_hbm.at[pl.ds((b * S + half * s_half) * 3 * nsample,
                                  s_half * 3 * nsample)])
        return

        def row_fn(s_local, carry):
            s_glob = half * s_half + s_local
            sidx = jnp.full((16,), s_glob, jnp.int32)
            cx = plsc.load_gather(cx_v, [sidx])
            cy = plsc.load_gather(cy_v, [sidx])
            cz = plsc.load_gather(cz_v, [sidx])
            cn = cx * cx + cy * cy + cz * cz

            def chunk_fn(i, kcnt):
                base = i * 16
                xv = x_v[pl.ds(base, 16)]
                yv = y_v[pl.ds(base, 16)]
                zv = z_v[pl.ds(base, 16)]
                nv = xn_v[pl.ds(base, 16)]
                d = -2.0 * (xv * cx + yv * cy + zv * cz) + nv + cn
                m = d <= r2
                ones = jnp.where(m, 1, 0).astype(jnp.int32)
                csum = plsc.cumsum(ones)
                tgt = kcnt + csum - 1
                mm = m & (tgt < cap - 16)
                plsc.store_scatter(idx_v, [tgt], base + iota16, mask=mm)
                return kcnt + jnp.sum(ones)

            kcnt = lax.fori_loop(0, n_chunks, chunk_fn, jnp.int32(0))

            i0 = plsc.load_gather(idx_v, [zero16])
            gbase = s_local * 3 * nsample
            for cc in range(nsample // 16):
                v = idx_v[pl.ds(cc * 16, 16)]
                pos = iota16 + (cc * 16)
                v = jnp.where(pos < kcnt, v, i0)
                gx = plsc.load_gather(x_v, [v])
                gy = plsc.load_gather(y_v, [v])
                gz = plsc.load_gather(z_v, [v])
                g_v[pl.ds(gbase + cc * 16, 16)] = gx - cx
                g_v[pl.ds(gbase + nsample + cc * 16, 16)] = gy - cy
                g_v[pl.ds(gbase + 2 * nsample + cc * 16, 16)] = gz - cz
            return carry
        lax.fori_loop(0, s_half, row_fn, jnp.int32(0))
        pltpu.sync_copy(
            g_v, out_hbm.at[pl.ds((b * S + half * s_half) * 3 * nsample,
                                  s_half * 3 * nsample)])

    return k(xyzt, cent)


def _query_ball_ref(radius, nsample, xyz, new_xyz):
    B, N, _ = xyz.shape
    S = new_xyz.shape[1]
    d = -2.0 * jnp.einsum('bsc,bnc->bsn', new_xyz, xyz)
    d = d + jnp.sum(new_xyz ** 2, axis=-1)[:, :, None]
    d = d + jnp.sum(xyz ** 2, axis=-1)[:, None, :]
    group_idx = jnp.broadcast_to(jnp.arange(N, dtype=jnp.int32), (B, S, N))
    group_idx = jnp.where(d > radius ** 2, N, group_idx)
    group_idx = jnp.sort(group_idx, axis=-1)[:, :, :nsample]
    group_first = jnp.broadcast_to(group_idx[:, :, :1], group_idx.shape)
    group_idx = jnp.where(group_idx == N, group_first, group_idx)
    return group_idx


def _tail_kernel(xyz2_ref, p2_ref, W3_0_ref, b3_0_ref, W3_1_ref, b3_1_ref,
                 Wd1_ref, bd1_ref, Wd3_ref, bd3_ref, out_ref):
    B, S, _ = p2_ref.shape
    feats = jnp.concatenate([xyz2_ref[...], p2_ref[...]], axis=-1)
    h = feats.reshape(B * S, feats.shape[-1])
    h = jnp.maximum(jnp.dot(h, W3_0_ref[...],
                            preferred_element_type=jnp.float32) + b3_0_ref[...], 0.0)
    h = jnp.maximum(jnp.dot(h, W3_1_ref[...],
                            preferred_element_type=jnp.float32) + b3_1_ref[...], 0.0)
    h = jnp.max(h.reshape(B, S, -1), axis=1)
    h = jnp.maximum(jnp.dot(h, Wd1_ref[...],
                            preferred_element_type=jnp.float32) + bd1_ref[...], 0.0)
    z = jnp.dot(h, Wd3_ref[...], preferred_element_type=jnp.float32) + bd3_ref[...]
    out_ref[...] = jax.nn.sigmoid(z)


def kernel(input, W1_0, b1_0, W1_1, b1_1, W2_0, b2_0, W2_1, b2_1,
           W3_0, b3_0, W3_1, b3_1, Wd1, bd1, Wd3, bd3):
    x = input
    B, N, _ = x.shape

    xb = x[:, :, 0]
    yb = x[:, :, 1]
    zb = x[:, :, 2]
    cx1, cy1, cz1 = _fps_tc(xb, yb, zb, 256)
    new_xyz = jnp.stack([cx1, cy1, cz1], axis=-1)
    xyzt = jnp.stack([xb, yb, zb], axis=1).reshape(-1)
    cent = jnp.stack([cx1, cy1, cz1], axis=1).reshape(-1)
    g1 = _ball_group_sc(xyzt, cent, 0.1, 32, N // 16, 128)
    g1 = g1.reshape(B, 256, 3, 32)
    np1 = jnp.transpose(g1, (0, 1, 3, 2))
    p1 = _sa_mlp(np1, [W1_0, W1_1], [b1_0, b1_1])

    cx2, cy2, cz2 = _fps_tc(cx1, cy1, cz1, 128)
    xyz2 = jnp.stack([cx2, cy2, cz2], axis=-1)
    idx2 = _query_ball_ref(0.2, 64, new_xyz, xyz2)
    g2 = _index_points(new_xyz, idx2) - xyz2[:, :, None, :]
    gp2 = _index_points(p1, idx2)
    np2 = jnp.concatenate([g2, gp2], axis=-1)
    p2 = _sa_mlp(np2, [W2_0, W2_1], [b2_0, b2_1])

    pred = pl.pallas_call(
        _tail_kernel,
        out_shape=jax.ShapeDtypeStruct((B, 1), jnp.float32),
    )(xyz2, p2, W3_0, b3_0, W3_1, b3_1, Wd1, bd1, Wd3, bd3)
    return pred


# speedup vs baseline: 10.1902x; 2.0200x over previous
import jax
import jax.numpy as jnp
from jax import lax
from jax.experimental import pallas as pl
from jax.experimental.pallas import tpu as pltpu
from jax.experimental.pallas import tpu_sc as plsc
from functools import partial


def _fps(xyz, npoint):
    B, N, _ = xyz.shape
    def body(i, state):
        centroids, distance, farthest = state
        centroids = centroids.at[:, i].set(farthest)
        centroid = jax.vmap(lambda p, j: p[j])(xyz, farthest)[:, None, :]
        dist = jnp.sum((xyz - centroid) ** 2, axis=-1)
        distance = jnp.minimum(distance, dist)
        farthest = jnp.argmax(distance, axis=-1).astype(jnp.int32)
        return (centroids, distance, farthest)
    init = (jnp.zeros((B, npoint), dtype=jnp.int32),
            jnp.full((B, N), 1e10, dtype=xyz.dtype),
            jnp.zeros((B,), dtype=jnp.int32))
    centroids, _, _ = jax.lax.fori_loop(0, npoint, body, init)
    return centroids


def _index_points(points, idx):
    return jax.vmap(lambda p, i: p[i])(points, idx)


def _fps_kernel_body(npoint, x_ref, y_ref, z_ref, cx_ref, cy_ref, cz_ref,
                     dist_ref):
    B, N = x_ref.shape
    xr = x_ref[...]
    yr = y_ref[...]
    zr = z_ref[...]
    iota_n = lax.broadcasted_iota(jnp.int32, (B, N), 1)
    iota_p = lax.broadcasted_iota(jnp.int32, (B, npoint), 1)
    dist_ref[...] = jnp.full((B, N), 1e10, jnp.float32)

    def step(i, carry):
        far, cxa, cya, cza = carry
        onehot = iota_n == far
        fx = jnp.sum(jnp.where(onehot, xr, 0.0), axis=1, keepdims=True)
        fy = jnp.sum(jnp.where(onehot, yr, 0.0), axis=1, keepdims=True)
        fz = jnp.sum(jnp.where(onehot, zr, 0.0), axis=1, keepdims=True)
        slot = iota_p == i
        cxa = jnp.where(slot, fx, cxa)
        cya = jnp.where(slot, fy, cya)
        cza = jnp.where(slot, fz, cza)
        d = (xr - fx) ** 2 + (yr - fy) ** 2 + (zr - fz) ** 2
        dist = jnp.minimum(dist_ref[...], d)
        dist_ref[...] = dist
        mx = jnp.max(dist, axis=1, keepdims=True)
        idx = jnp.min(jnp.where(dist == mx, iota_n, N), axis=1, keepdims=True)
        return (idx.astype(jnp.int32), cxa, cya, cza)

    zc = jnp.zeros((B, npoint), jnp.float32)
    _, cxa, cya, cza = lax.fori_loop(
        0, npoint, step, (jnp.zeros((B, 1), jnp.int32), zc, zc, zc))
    cx_ref[...] = cxa
    cy_ref[...] = cya
    cz_ref[...] = cza


def _fps_tc(xb, yb, zb, npoint):
    B, N = xb.shape
    out = pl.pallas_call(
        partial(_fps_kernel_body, npoint),
        out_shape=[jax.ShapeDtypeStruct((B, npoint), jnp.float32)] * 3,
        scratch_shapes=[pltpu.VMEM((B, N), jnp.float32)],
    )(xb, yb, zb)
    return out


def _sa_mlp(new_points, Ws, bs):
    for W, b in zip(Ws, bs):
        new_points = jax.nn.relu(new_points @ W + b)
    return jnp.max(new_points, axis=2)


def _ball_group_sc(xyzt, cent, radius, nsample, n_chunks, s_half,
                   feats=None, nfeat=0):
    B3N = xyzt.shape[0]
    N = n_chunks * 16
    S = 2 * s_half
    B = B3N // (3 * N)
    C = 3 + nfeat
    r2 = jnp.float32(float(radius) ** 2)
    cap = 2 * nsample
    nf = max(nfeat, 1)
    if feats is None:
        feats = jnp.zeros((16,), jnp.float32)
        fvn = 16
    else:
        fvn = nf * N
    mesh = plsc.VectorSubcoreMesh(core_axis_name="c", subcore_axis_name="s")

    @partial(
        pl.kernel,
        mesh=mesh,
        compiler_params=pltpu.CompilerParams(needs_layout_passes=False),
        out_type=jax.ShapeDtypeStruct((B * S * C * nsample,), jnp.float32),
        scratch_types=[
            pltpu.VMEM((N,), jnp.float32),
            pltpu.VMEM((N,), jnp.float32),
            pltpu.VMEM((N,), jnp.float32),
            pltpu.VMEM((N,), jnp.float32),
            pltpu.VMEM((S,), jnp.float32),
            pltpu.VMEM((S,), jnp.float32),
            pltpu.VMEM((S,), jnp.float32),
            pltpu.VMEM((cap,), jnp.float32),
            pltpu.VMEM((cap,), jnp.float32),
            pltpu.VMEM((cap,), jnp.float32),
            pltpu.VMEM((fvn,), jnp.float32),
            pltpu.VMEM((nf * cap,), jnp.float32),
            pltpu.VMEM((s_half * C * nsample,), jnp.float32),
        ],
    )
    def k(xyzt_hbm, cent_hbm, feats_hbm, out_hbm, x_v, y_v, z_v, xn_v,
          cx_v, cy_v, cz_v, gx_b, gy_b, gz_b, f_v, gf_b, g_v):
        cid = lax.axis_index("c")
        sid = lax.axis_index("s")
        wid = sid * 2 + cid
        b = wid // 2
        half = wid % 2
        pltpu.sync_copy(xyzt_hbm.at[pl.ds(b * 3 * N, N)], x_v)
        pltpu.sync_copy(xyzt_hbm.at[pl.ds(b * 3 * N + N, N)], y_v)
        pltpu.sync_copy(xyzt_hbm.at[pl.ds(b * 3 * N + 2 * N, N)], z_v)
        pltpu.sync_copy(cent_hbm.at[pl.ds(b * 3 * S, S)], cx_v)
        pltpu.sync_copy(cent_hbm.at[pl.ds(b * 3 * S + S, S)], cy_v)
        pltpu.sync_copy(cent_hbm.at[pl.ds(b * 3 * S + 2 * S, S)], cz_v)
        if nfeat:
            pltpu.sync_copy(feats_hbm.at[pl.ds(b * nfeat * N, nfeat * N)],
                            f_v)

        iota16 = lax.iota(jnp.int32, 16)

        def norm_fn(i, carry):
            base = i * 16
            xv = x_v[pl.ds(base, 16)]
            yv = y_v[pl.ds(base, 16)]
            zv = z_v[pl.ds(base, 16)]
            xn_v[pl.ds(base, 16)] = xv * xv + yv * yv + zv * zv
            return carry
        lax.fori_loop(0, n_chunks, norm_fn, jnp.int32(0))

        def t2_row(s_local, carry):
            s_glob = half * s_half + s_local
            sidx = jnp.full((16,), s_glob, jnp.int32)
            cx = plsc.load_gather(cx_v, [sidx])
            cy = plsc.load_gather(cy_v, [sidx])
            cz = plsc.load_gather(cz_v, [sidx])
            cn = cx * cx + cy * cy + cz * cz
            iota16_ = lax.iota(jnp.int32, 16)

            def chunk_fn(i, kcnt):
                base = i * 16
                xv = x_v[pl.ds(base, 16)]
                yv = y_v[pl.ds(base, 16)]
                zv = z_v[pl.ds(base, 16)]
                nv = xn_v[pl.ds(base, 16)]
                d = -2.0 * (xv * cx + yv * cy + zv * cz) + nv + cn
                m = d <= r2
                ones = jnp.where(m, 1, 0).astype(jnp.int32)
                csum = plsc.cumsum(ones)
                tgt = kcnt + csum - 1
                mm = m & (tgt < cap - 16)
                plsc.store_scatter(gx_b, [tgt], xv - cx, mask=mm)
                plsc.store_scatter(gy_b, [tgt], yv - cy, mask=mm)
                plsc.store_scatter(gz_b, [tgt], zv - cz, mask=mm)
                for r in range(nfeat):
                    fv = f_v[pl.ds(r * N + base, 16)]
                    plsc.store_scatter(gf_b, [tgt + r * cap], fv, mask=mm)
                return kcnt + jnp.sum(ones)

            kcnt = lax.fori_loop(0, n_chunks, chunk_fn, jnp.int32(0))
            gbase = s_local * C * nsample
            zidx = jnp.zeros((16,), jnp.int32)
            g0x = plsc.load_gather(gx_b, [zidx])
            g0y = plsc.load_gather(gy_b, [zidx])
            g0z = plsc.load_gather(gz_b, [zidx])
            for cc in range(nsample // 16):
                pos = iota16_ + (cc * 16)
                keep = pos < kcnt
                vx = jnp.where(keep, gx_b[pl.ds(cc * 16, 16)], g0x)
                vy = jnp.where(keep, gy_b[pl.ds(cc * 16, 16)], g0y)
                vz = jnp.where(keep, gz_b[pl.ds(cc * 16, 16)], g0z)
                g_v[pl.ds(gbase + cc * 16, 16)] = vx
                g_v[pl.ds(gbase + nsample + cc * 16, 16)] = vy
                g_v[pl.ds(gbase + 2 * nsample + cc * 16, 16)] = vz
            for r in range(nfeat):
                g0f = plsc.load_gather(
                    gf_b, [jnp.full((16,), r * cap, jnp.int32)])
                for cc in range(nsample // 16):
                    pos = iota16_ + (cc * 16)
                    keep = pos < kcnt
                    vf = jnp.where(
                        keep, gf_b[pl.ds(r * cap + cc * 16, 16)], g0f)
                    g_v[pl.ds(gbase + (3 + r) * nsample + cc * 16, 16)] = vf
            return carry
        lax.fori_loop(0, s_half, t2_row, jnp.int32(0))
        pltpu.sync_copy(
            g_v, out_hbm.at[pl.ds((b * S + half * s_half) * C * nsample,
                                  s_half * C * nsample)])
        return

    return k(xyzt, cent, feats)


def _query_ball_ref(radius, nsample, xyz, new_xyz):
    B, N, _ = xyz.shape
    S = new_xyz.shape[1]
    d = -2.0 * jnp.einsum('bsc,bnc->bsn', new_xyz, xyz)
    d = d + jnp.sum(new_xyz ** 2, axis=-1)[:, :, None]
    d = d + jnp.sum(xyz ** 2, axis=-1)[:, None, :]
    group_idx = jnp.broadcast_to(jnp.arange(N, dtype=jnp.int32), (B, S, N))
    group_idx = jnp.where(d > radius ** 2, N, group_idx)
    group_idx = jnp.sort(group_idx, axis=-1)[:, :, :nsample]
    group_first = jnp.broadcast_to(group_idx[:, :, :1], group_idx.shape)
    group_idx = jnp.where(group_idx == N, group_first, group_idx)
    return group_idx


def _tail_kernel(xyz2_ref, p2_ref, W3_0_ref, b3_0_ref, W3_1_ref, b3_1_ref,
                 Wd1_ref, bd1_ref, Wd3_ref, bd3_ref, out_ref):
    B, S, _ = p2_ref.shape
    feats = jnp.concatenate([xyz2_ref[...], p2_ref[...]], axis=-1)
    h = feats.reshape(B * S, feats.shape[-1])
    h = jnp.maximum(jnp.dot(h, W3_0_ref[...],
                            preferred_element_type=jnp.float32) + b3_0_ref[...], 0.0)
    h = jnp.maximum(jnp.dot(h, W3_1_ref[...],
                            preferred_element_type=jnp.float32) + b3_1_ref[...], 0.0)
    h = jnp.max(h.reshape(B, S, -1), axis=1)
    h = jnp.maximum(jnp.dot(h, Wd1_ref[...],
                            preferred_element_type=jnp.float32) + bd1_ref[...], 0.0)
    z = jnp.dot(h, Wd3_ref[...], preferred_element_type=jnp.float32) + bd3_ref[...]
    out_ref[...] = jax.nn.sigmoid(z)


def kernel(input, W1_0, b1_0, W1_1, b1_1, W2_0, b2_0, W2_1, b2_1,
           W3_0, b3_0, W3_1, b3_1, Wd1, bd1, Wd3, bd3):
    x = input
    B, N, _ = x.shape

    xb = x[:, :, 0]
    yb = x[:, :, 1]
    zb = x[:, :, 2]
    cx1, cy1, cz1 = _fps_tc(xb, yb, zb, 256)
    new_xyz = jnp.stack([cx1, cy1, cz1], axis=-1)
    xyzt = jnp.stack([xb, yb, zb], axis=1).reshape(-1)
    cent = jnp.stack([cx1, cy1, cz1], axis=1).reshape(-1)
    g1 = _ball_group_sc(xyzt, cent, 0.1, 32, N // 16, 128)
    g1 = g1.reshape(B, 256, 3, 32)
    np1 = jnp.transpose(g1, (0, 1, 3, 2))
    p1 = _sa_mlp(np1, [W1_0, W1_1], [b1_0, b1_1])

    cx2, cy2, cz2 = _fps_tc(cx1, cy1, cz1, 128)
    xyz2 = jnp.stack([cx2, cy2, cz2], axis=-1)
    cent2 = jnp.stack([cx2, cy2, cz2], axis=1).reshape(-1)
    p1t = jnp.transpose(p1, (0, 2, 1)).reshape(-1)
    g2 = _ball_group_sc(cent, cent2, 0.2, 64, 256 // 16, 64,
                        feats=p1t, nfeat=16)
    g2 = g2.reshape(B, 128, 19, 64)
    np2 = jnp.transpose(g2, (0, 1, 3, 2))
    p2 = _sa_mlp(np2, [W2_0, W2_1], [b2_0, b2_1])

    pred = pl.pallas_call(
        _tail_kernel,
        out_shape=jax.ShapeDtypeStruct((B, 1), jnp.float32),
    )(xyz2, p2, W3_0, b3_0, W3_1, b3_1, Wd1, bd1, Wd3, bd3)
    return pred
